# baseline (device time: 25884 ns/iter reference)
import jax
import jax.numpy as jnp
from jax import lax
from jax.experimental import pallas as pl
from jax.experimental.pallas import tpu as pltpu

CHUNK_ROWS = (256, 256, 192, 128, 96, 48, 32, 16)
K = len(CHUNK_ROWS)
CHUNK_OFFS = tuple(sum(CHUNK_ROWS[:i]) for i in range(K))


def kernel(x):
    m, n = x.shape
    nh = n // 2
    mh = m // 2
    out_m = 2 * m
    assert sum(CHUNK_ROWS) == mh

    def body(x_ref, out_ref, send_buf, xsend_sems, xrecv_sems,
             ysend_sems, yrecv_sems):
        my_x = lax.axis_index("x")
        my_y = lax.axis_index("y")
        my_z = lax.axis_index("z")
        px = 1 - my_x
        ny = 1 - my_y

        send_buf[...] = x_ref[
            pl.ds(my_y * mh, mh), pl.ds(px * nh, nh)
        ].astype(jnp.bfloat16)
        out_ref[pl.ds(my_x * m, m), :] = x_ref[
            :, pl.ds(my_x * nh, nh)
        ].astype(jnp.bfloat16)

        barrier_sem = pltpu.get_barrier_semaphore()
        pl.semaphore_signal(
            barrier_sem, inc=1, device_id=(px, my_y, my_z),
            device_id_type=pl.DeviceIdType.MESH,
        )
        pl.semaphore_signal(
            barrier_sem, inc=1, device_id=(my_x, ny, my_z),
            device_id_type=pl.DeviceIdType.MESH,
        )
        pl.semaphore_wait(barrier_sem, 2)

        send_base = my_x * m + my_y * mh
        recv_base = px * m + my_y * mh

        x_rdmas = []
        for i in range(K):
            r = pltpu.make_async_remote_copy(
                src_ref=send_buf.at[pl.ds(CHUNK_OFFS[i], CHUNK_ROWS[i]), :],
                dst_ref=out_ref.at[
                    pl.ds(send_base + CHUNK_OFFS[i], CHUNK_ROWS[i]), :
                ],
                send_sem=xsend_sems.at[i],
                recv_sem=xrecv_sems.at[i],
                device_id=(px, my_y, my_z),
                device_id_type=pl.DeviceIdType.MESH,
            )
            r.start()
            x_rdmas.append(r)

        y_rdmas = []
        for i in range(K):
            x_rdmas[i].wait_recv()
            r = pltpu.make_async_remote_copy(
                src_ref=out_ref.at[
                    pl.ds(recv_base + CHUNK_OFFS[i], CHUNK_ROWS[i]), :
                ],
                dst_ref=out_ref.at[
                    pl.ds(recv_base + CHUNK_OFFS[i], CHUNK_ROWS[i]), :
                ],
                send_sem=ysend_sems.at[i],
                recv_sem=yrecv_sems.at[i],
                device_id=(my_x, ny, my_z),
                device_id_type=pl.DeviceIdType.MESH,
            )
            r.start()
            y_rdmas.append(r)

        for i in range(K):
            y_rdmas[i].wait_recv()
            x_rdmas[i].wait_send()
            y_rdmas[i].wait_send()

    return pl.pallas_call(
        body,
        out_shape=jax.ShapeDtypeStruct((out_m, nh), jnp.bfloat16),
        in_specs=[pl.BlockSpec(memory_space=pltpu.VMEM)],
        out_specs=pl.BlockSpec(memory_space=pltpu.VMEM),
        scratch_shapes=[
            pltpu.VMEM((mh, nh), jnp.bfloat16),
            pltpu.SemaphoreType.DMA((K,)),
            pltpu.SemaphoreType.DMA((K,)),
            pltpu.SemaphoreType.DMA((K,)),
            pltpu.SemaphoreType.DMA((K,)),
        ],
        compiler_params=pltpu.CompilerParams(collective_id=0),
    )(x)


# device time: 25614 ns/iter; 1.0105x vs baseline; 1.0105x over previous
import jax
import jax.numpy as jnp
from jax import lax
from jax.experimental import pallas as pl
from jax.experimental.pallas import tpu as pltpu

CHUNK_ROWS = (256, 256, 192, 128, 96, 48, 32, 16)
K = len(CHUNK_ROWS)
CHUNK_OFFS = tuple(sum(CHUNK_ROWS[:i]) for i in range(K))


def kernel(x):
    m, n = x.shape
    nh = n // 2
    mh = m // 2
    out_m = 2 * m
    assert sum(CHUNK_ROWS) == mh

    def body(x_ref, out_ref, send_buf, xsend_sems, xrecv_sems,
             ysend_sems, yrecv_sems):
        my_x = lax.axis_index("x")
        my_y = lax.axis_index("y")
        my_z = lax.axis_index("z")
        px = 1 - my_x
        ny = 1 - my_y

        barrier_sem = pltpu.get_barrier_semaphore()
        pl.semaphore_signal(
            barrier_sem, inc=1, device_id=(px, my_y, my_z),
            device_id_type=pl.DeviceIdType.MESH,
        )
        pl.semaphore_signal(
            barrier_sem, inc=1, device_id=(my_x, ny, my_z),
            device_id_type=pl.DeviceIdType.MESH,
        )
        pl.semaphore_wait(barrier_sem, 2)

        send_base = my_x * m + my_y * mh
        recv_base = px * m + my_y * mh

        x_rdmas = []
        for i in range(K):
            send_buf[pl.ds(CHUNK_OFFS[i], CHUNK_ROWS[i]), :] = x_ref[
                pl.ds(my_y * mh + CHUNK_OFFS[i], CHUNK_ROWS[i]),
                pl.ds(px * nh, nh),
            ].astype(jnp.bfloat16)
            r = pltpu.make_async_remote_copy(
                src_ref=send_buf.at[pl.ds(CHUNK_OFFS[i], CHUNK_ROWS[i]), :],
                dst_ref=out_ref.at[
                    pl.ds(send_base + CHUNK_OFFS[i], CHUNK_ROWS[i]), :
                ],
                send_sem=xsend_sems.at[i],
                recv_sem=xrecv_sems.at[i],
                device_id=(px, my_y, my_z),
                device_id_type=pl.DeviceIdType.MESH,
            )
            r.start()
            x_rdmas.append(r)

        out_ref[pl.ds(my_x * m, m), :] = x_ref[
            :, pl.ds(my_x * nh, nh)
        ].astype(jnp.bfloat16)

        y_rdmas = []
        for i in range(K):
            x_rdmas[i].wait_recv()
            r = pltpu.make_async_remote_copy(
                src_ref=out_ref.at[
                    pl.ds(recv_base + CHUNK_OFFS[i], CHUNK_ROWS[i]), :
                ],
                dst_ref=out_ref.at[
                    pl.ds(recv_base + CHUNK_OFFS[i], CHUNK_ROWS[i]), :
                ],
                send_sem=ysend_sems.at[i],
                recv_sem=yrecv_sems.at[i],
                device_id=(my_x, ny, my_z),
                device_id_type=pl.DeviceIdType.MESH,
            )
            r.start()
            y_rdmas.append(r)

        for i in range(K):
            y_rdmas[i].wait_recv()
            x_rdmas[i].wait_send()
            y_rdmas[i].wait_send()

    return pl.pallas_call(
        body,
        out_shape=jax.ShapeDtypeStruct((out_m, nh), jnp.bfloat16),
        in_specs=[pl.BlockSpec(memory_space=pltpu.VMEM)],
        out_specs=pl.BlockSpec(memory_space=pltpu.VMEM),
        scratch_shapes=[
            pltpu.VMEM((mh, nh), jnp.bfloat16),
            pltpu.SemaphoreType.DMA((K,)),
            pltpu.SemaphoreType.DMA((K,)),
            pltpu.SemaphoreType.DMA((K,)),
            pltpu.SemaphoreType.DMA((K,)),
        ],
        compiler_params=pltpu.CompilerParams(collective_id=0),
    )(x)
